# Initial kernel scaffold; baseline (speedup 1.0000x reference)
import jax
import jax.numpy as jnp
from jax.experimental import pallas as pl


def kernel(x, edge_index, edge_attr, batch, W0, b0, W1, b1, W2, b2, Wr0, br0, Wr1, br1):
    raise NotImplementedError("write your pallas kernel here")


# speedup vs baseline: 6.0285x; 6.0285x over previous
import functools

import jax
import jax.numpy as jnp
from jax import lax
from jax.experimental import pallas as pl
from jax.experimental.pallas import tpu as pltpu
from jax.experimental.pallas import tpu_sc as plsc

N_NODES = 10000
N_PAD = 10240
EDGES = 320000
D = 128
NG = 64
NC, NS, L = 2, 16, 16
NW = NC * NS
CHUNK = 128
CHUNKS_PER_TILE = 80
EPT = CHUNKS_PER_TILE * CHUNK
E_PAD = NW * EPT
ROWS_PER_TILE = N_PAD // NS
BLK = 1280
GRID = N_PAD // BLK


def _deg_body(col_hbm, w_hbm, out_hbm, col_v, w_v, deg_v):
    c = lax.axis_index("c")
    s = lax.axis_index("s")
    wid = s * NC + c
    base = wid * EPT
    pltpu.sync_copy(col_hbm.at[pl.ds(base, EPT)], col_v)
    pltpu.sync_copy(w_hbm.at[pl.ds(base, EPT)], w_v)
    zv = jnp.zeros((L,), jnp.float32)

    def zbody(i, _):
        deg_v[pl.ds(i * L, L)] = zv
        return 0

    lax.fori_loop(0, N_PAD // L, zbody, 0)
    lanes = lax.broadcasted_iota(jnp.int32, (L,), 0)

    def kbody(k, _):
        cvec = col_v[pl.ds(k * L, L)]
        wvec = w_v[pl.ds(k * L, L)]
        for e in range(L):
            cc = cvec[e]
            base = (cc >> 4) << 4
            oh = jnp.where(lanes == (cc - base), wvec[e], 0.0)
            plsc.addupdate(deg_v.at[pl.ds(base, L)], oh)
        return 0

    lax.fori_loop(0, EPT // L, kbody, 0)
    pltpu.sync_copy(deg_v, out_hbm.at[wid])


_deg_pass = pl.kernel(
    _deg_body,
    out_type=jax.ShapeDtypeStruct((NW, N_PAD), jnp.float32),
    mesh=plsc.VectorSubcoreMesh(core_axis_name="c", subcore_axis_name="s"),
    scratch_types=[
        pltpu.VMEM((EPT,), jnp.int32),
        pltpu.VMEM((EPT,), jnp.float32),
        pltpu.VMEM((N_PAD,), jnp.float32),
    ],
)


def _edge_body(g_hbm, row_hbm, col_hbm, w_hbm, out_hbm,
               row_v, col_v, w_v, buf, acc_sh):
    c = lax.axis_index("c")
    s = lax.axis_index("s")
    wid = s * NC + c
    pltpu.sync_copy(row_hbm.at[wid], row_v)
    pltpu.sync_copy(col_hbm.at[wid], col_v)
    pltpu.sync_copy(w_hbm.at[wid], w_v)

    zv = jnp.zeros((L,), jnp.float32)

    def zbody(i, _):
        for q in range(D // L):
            buf[i, pl.ds(q * L, L)] = zv
        return 0

    lax.fori_loop(0, CHUNK, zbody, 0)
    for k in range(ROWS_PER_TILE // CHUNK):
        pltpu.sync_copy(buf, acc_sh.at[pl.ds(s * ROWS_PER_TILE + k * CHUNK, CHUNK)])
    plsc.subcore_barrier()

    def chunk_body(j, _):
        pltpu.sync_copy(g_hbm.at[row_v.at[j]], buf)

        def mul_body(k, _):
            wvec = w_v[j, pl.ds(k * L, L)]
            for e in range(L):
                wv = wvec[e]
                idx = k * L + e
                for q in range(D // L):
                    sl = pl.ds(q * L, L)
                    buf[idx, sl] = buf[idx, sl] * wv
            return 0

        lax.fori_loop(0, CHUNK // L, mul_body, 0)
        pltpu.sync_copy(buf, acc_sh.at[col_v.at[j]], add=True)
        return 0

    lax.fori_loop(0, CHUNKS_PER_TILE, chunk_body, 0)
    plsc.subcore_barrier()
    for k in range(ROWS_PER_TILE // CHUNK):
        off = s * ROWS_PER_TILE + k * CHUNK
        pltpu.sync_copy(acc_sh.at[pl.ds(off, CHUNK)],
                        out_hbm.at[c, pl.ds(off, CHUNK)])


_edge_pass = pl.kernel(
    _edge_body,
    out_type=jax.ShapeDtypeStruct((NC, N_PAD, D), jnp.float32),
    mesh=plsc.VectorSubcoreMesh(core_axis_name="c", subcore_axis_name="s"),
    scratch_types=[
        pltpu.VMEM((CHUNKS_PER_TILE, CHUNK), jnp.int32),
        pltpu.VMEM((CHUNKS_PER_TILE, CHUNK), jnp.int32),
        pltpu.VMEM((CHUNKS_PER_TILE, CHUNK), jnp.float32),
        pltpu.VMEM((CHUNK, D), jnp.float32),
        pltpu.VMEM_SHARED((N_PAD, D), jnp.float32),
    ],
)


def _tc1_body(degp, x, w0, dinv_ref, g_ref):
    d = jnp.sum(degp[...], axis=0) + 1.0
    dv = lax.rsqrt(d)[:, None]
    dinv_ref[...] = dv
    g_ref[...] = jnp.dot(x[...], w0[...],
                         preferred_element_type=jnp.float32) * dv


_tc1 = pl.pallas_call(
    _tc1_body,
    grid=(GRID,),
    in_specs=[
        pl.BlockSpec((NW, BLK), lambda i: (0, i)),
        pl.BlockSpec((BLK, D), lambda i: (i, 0)),
        pl.BlockSpec((D, D), lambda i: (0, 0)),
    ],
    out_specs=[
        pl.BlockSpec((BLK, 1), lambda i: (i, 0)),
        pl.BlockSpec((BLK, D), lambda i: (i, 0)),
    ],
    out_shape=[
        jax.ShapeDtypeStruct((N_PAD, 1), jnp.float32),
        jax.ShapeDtypeStruct((N_PAD, D), jnp.float32),
    ],
)


def _tc_mid_body(s0, s1, g, dinv, b, w, out_ref):
    dv = dinv[...]
    ssum = s0[...] + s1[...] + g[...]
    h = jnp.maximum(ssum * dv + b[...], 0.0)
    out_ref[...] = jnp.dot(h, w[...],
                           preferred_element_type=jnp.float32) * dv


_tc_mid = pl.pallas_call(
    _tc_mid_body,
    grid=(GRID,),
    in_specs=[
        pl.BlockSpec((BLK, D), lambda i: (i, 0)),
        pl.BlockSpec((BLK, D), lambda i: (i, 0)),
        pl.BlockSpec((BLK, D), lambda i: (i, 0)),
        pl.BlockSpec((BLK, 1), lambda i: (i, 0)),
        pl.BlockSpec((1, D), lambda i: (0, 0)),
        pl.BlockSpec((D, D), lambda i: (0, 0)),
    ],
    out_specs=pl.BlockSpec((BLK, D), lambda i: (i, 0)),
    out_shape=jax.ShapeDtypeStruct((N_PAD, D), jnp.float32),
)


def _tc4_body(s0, s1, g, dinv, b, wr0, br0, wr1, br1, batch,
              out_ref, acc_s, acc_c):
    i = pl.program_id(0)
    ssum = s0[...] + s1[...] + g[...]
    h = ssum * dinv[...] + b[...]
    t = jnp.maximum(jnp.dot(h, wr0[...],
                            preferred_element_type=jnp.float32) + br0[...], 0.0)
    r = jnp.dot(t, wr1[...], preferred_element_type=jnp.float32) + br1[...]
    oh = (batch[...] == lax.broadcasted_iota(jnp.int32, (1, NG), 1)
          ).astype(jnp.float32)
    ps = jnp.sum(oh * r, axis=0, keepdims=True)
    pc = jnp.sum(oh, axis=0, keepdims=True)

    @pl.when(i == 0)
    def _():
        acc_s[...] = ps
        acc_c[...] = pc

    @pl.when(i > 0)
    def _():
        acc_s[...] += ps
        acc_c[...] += pc

    @pl.when(i == pl.num_programs(0) - 1)
    def _():
        out_ref[...] = acc_s[...] / jnp.maximum(acc_c[...], 1.0)


_tc4 = pl.pallas_call(
    _tc4_body,
    grid=(GRID,),
    in_specs=[
        pl.BlockSpec((BLK, D), lambda i: (i, 0)),
        pl.BlockSpec((BLK, D), lambda i: (i, 0)),
        pl.BlockSpec((BLK, D), lambda i: (i, 0)),
        pl.BlockSpec((BLK, 1), lambda i: (i, 0)),
        pl.BlockSpec((1, D), lambda i: (0, 0)),
        pl.BlockSpec((D, NG), lambda i: (0, 0)),
        pl.BlockSpec((1, NG), lambda i: (0, 0)),
        pl.BlockSpec((NG, 1), lambda i: (0, 0)),
        pl.BlockSpec((1, 1), lambda i: (0, 0)),
        pl.BlockSpec((BLK, 1), lambda i: (i, 0)),
    ],
    out_specs=pl.BlockSpec((1, NG), lambda i: (0, 0)),
    out_shape=jax.ShapeDtypeStruct((1, NG), jnp.float32),
    scratch_shapes=[
        pltpu.VMEM((1, NG), jnp.float32),
        pltpu.VMEM((1, NG), jnp.float32),
    ],
)


@jax.jit
def kernel(x, edge_index, edge_attr, batch,
           W0, b0, W1, b1, W2, b2, Wr0, br0, Wr1, br1):
    pad_e = E_PAD - EDGES
    row_p = jnp.concatenate([edge_index[0],
                             jnp.zeros((pad_e,), jnp.int32)])
    col_p = jnp.concatenate([edge_index[1],
                             jnp.zeros((pad_e,), jnp.int32)])
    w_p = jnp.concatenate([edge_attr, jnp.zeros((pad_e,), jnp.float32)])
    row3 = row_p.reshape(NW, CHUNKS_PER_TILE, CHUNK)
    col3 = col_p.reshape(NW, CHUNKS_PER_TILE, CHUNK)
    w3 = w_p.reshape(NW, CHUNKS_PER_TILE, CHUNK)

    pad_n = N_PAD - N_NODES
    x_pad = jnp.concatenate([x, jnp.zeros((pad_n, D), jnp.float32)])
    batch_pad = jnp.concatenate(
        [batch, jnp.full((pad_n,), NG, jnp.int32)])[:, None]

    deg_parts = _deg_pass(col_p, w_p)
    dinv, g = _tc1(deg_parts, x_pad, W0)

    sp = _edge_pass(g, row3, col3, w3)
    g = _tc_mid(sp[0], sp[1], g, dinv, b0.reshape(1, D), W1)

    sp = _edge_pass(g, row3, col3, w3)
    g = _tc_mid(sp[0], sp[1], g, dinv, b1.reshape(1, D), W2)

    sp = _edge_pass(g, row3, col3, w3)
    out = _tc4(sp[0], sp[1], g, dinv, b2.reshape(1, D),
               Wr0, br0.reshape(1, NG), Wr1, br1.reshape(1, 1), batch_pad)
    return out.reshape(NG, 1)
